# baseline (speedup 1.0000x reference)
import functools

import jax
import jax.numpy as jnp
from jax import lax
from jax.experimental import pallas as pl
from jax.experimental.pallas import tpu as pltpu
from jax.experimental.pallas import tpu_sc as plsc

LANES = 16
CHUNK = 128
HALF = 64
N_SUB = 16


def _linear_body(x_ref, w_ref, b_ref, o_ref):
    h = lax.dot_general(
        x_ref[...], w_ref[...], (((1,), (1,)), ((), ())),
        preferred_element_type=jnp.float32) + b_ref[...][None, :]
    o_ref[0] = h[:, :HALF]
    o_ref[1] = h[:, HALF:]


def _linear_split(x, W, b):
    n, _ = x.shape
    return pl.pallas_call(
        _linear_body,
        out_shape=jax.ShapeDtypeStruct((2, n, HALF), jnp.float32),
    )(x, W, b)


def _edge_agg_body(nchunks_ps, nvalid_ps, n_nodes,
                   h1_hbm, srcm_hbm, dstm_hbm, acc_hbm, deg_hbm,
                   rows0_v, rows1_v, src_v, dst_v, zbuf_v, degl_v,
                   acc_sh, semg0, semg1):
    cid = lax.axis_index("c")
    sid = lax.axis_index("s")
    zrows = zbuf_v.shape[0]
    nzchunks = n_nodes // zrows
    zk = (nzchunks + N_SUB - 1) // N_SUB

    def _z2(i, _):
        r = i // (HALF // LANES)
        c = i % (HALF // LANES)
        zbuf_v[r, pl.ds(c * LANES, LANES)] = jnp.zeros((LANES,), jnp.float32)
        return 0
    lax.fori_loop(0, zrows * (HALF // LANES), _z2, 0)

    def _zacc(k, _):
        j = sid + k * N_SUB
        @pl.when(j < nzchunks)
        def _():
            pltpu.sync_copy(zbuf_v, acc_sh.at[pl.ds(j * zrows, zrows)])
        return 0
    lax.fori_loop(0, zk, _zacc, 0)

    pltpu.sync_copy(srcm_hbm.at[sid], src_v)
    pltpu.sync_copy(dstm_hbm.at[sid], dst_v)

    def _zd(i, _):
        degl_v[pl.ds(i * LANES, LANES)] = jnp.zeros((LANES,), jnp.float32)
        return 0
    lax.fori_loop(0, n_nodes // LANES, _zd, 0)

    ones16 = jnp.ones((LANES,), jnp.float32)
    vpc = CHUNK // LANES
    def _hist(i, _):
        idx = src_v[i // vpc, pl.ds((i % vpc) * LANES, LANES)]
        plsc.addupdate_scatter(degl_v, [idx], ones16)
        return 0
    lax.fori_loop(0, nvalid_ps // LANES, _hist, 0)
    pltpu.sync_copy(degl_v,
                    deg_hbm.at[pl.ds((cid * N_SUB + sid) * n_nodes, n_nodes)])

    plsc.subcore_barrier()

    table = h1_hbm.at[cid]
    pltpu.async_copy(table.at[src_v.at[0]], rows0_v, semg0)

    def _edge(i, _):
        j0 = 2 * i
        d1 = pltpu.async_copy(table.at[src_v.at[j0 + 1]], rows1_v, semg1)
        pltpu.make_async_copy(table.at[src_v.at[j0]], rows0_v, semg0).wait()
        pltpu.sync_copy(rows0_v, acc_sh.at[dst_v.at[j0]], add=True)
        jn = jnp.where(j0 + 2 < nchunks_ps, j0 + 2, 0)
        pltpu.async_copy(table.at[src_v.at[jn]], rows0_v, semg0)
        d1.wait()
        pltpu.sync_copy(rows1_v, acc_sh.at[dst_v.at[j0 + 1]], add=True)
        return 0
    lax.fori_loop(0, nchunks_ps // 2, _edge, 0)
    pltpu.make_async_copy(table.at[src_v.at[0]], rows0_v, semg0).wait()

    plsc.subcore_barrier()

    def _wacc(k, _):
        j = sid + k * N_SUB
        @pl.when(j < nzchunks)
        def _():
            off = j * zrows
            pltpu.sync_copy(acc_sh.at[pl.ds(off, zrows)], zbuf_v)
            pltpu.sync_copy(zbuf_v, acc_hbm.at[cid, pl.ds(off, zrows)])
        return 0
    lax.fori_loop(0, zk, _wacc, 0)


def _edge_aggregate(h1s, srcm, dstm, nvalid_ps, n_nodes):
    nchunks_ps = srcm.shape[1]
    mesh = plsc.VectorSubcoreMesh(core_axis_name="c", subcore_axis_name="s")
    kern = pl.kernel(
        functools.partial(_edge_agg_body, nchunks_ps, nvalid_ps, n_nodes),
        out_type=(
            jax.ShapeDtypeStruct((2, n_nodes, HALF), jnp.float32),
            jax.ShapeDtypeStruct((2 * N_SUB * n_nodes,), jnp.float32),
        ),
        mesh=mesh,
        compiler_params=pltpu.CompilerParams(use_tc_tiling_on_sc=False,
                                             needs_layout_passes=False),
        scratch_types=(
            pltpu.VMEM((CHUNK, HALF), jnp.float32),
            pltpu.VMEM((CHUNK, HALF), jnp.float32),
            pltpu.VMEM((nchunks_ps, CHUNK), jnp.int32),
            pltpu.VMEM((nchunks_ps, CHUNK), jnp.int32),
            pltpu.VMEM((200, HALF), jnp.float32),
            pltpu.VMEM((n_nodes,), jnp.float32),
            pltpu.VMEM_SHARED((n_nodes + 8, HALF), jnp.float32),
            pltpu.SemaphoreType.DMA,
            pltpu.SemaphoreType.DMA,
        ),
    )
    return kern(h1s, srcm, dstm)


def _combine_body(n_edges, acc_ref, deg_ref, w2_ref, b2_ref, o_ref):
    w2s = jnp.sum(w2_ref[...], axis=0)
    deg = 0.5 * jnp.sum(deg_ref[...], axis=0)
    ta = jnp.tanh(acc_ref[0])
    tb = jnp.tanh(acc_ref[1])
    row = jnp.sum(ta * w2s[None, :HALF], axis=1) + jnp.sum(tb * w2s[None, HALF:], axis=1)
    total = jnp.sum(row * deg) + n_edges * jnp.sum(b2_ref[...])
    o_ref[...] = total[None, None]


def _combine(acc, deg, W2, b2, n_edges):
    return pl.pallas_call(
        functools.partial(_combine_body, float(n_edges)),
        out_shape=jax.ShapeDtypeStruct((1, 1), jnp.float32),
    )(acc, deg, W2, b2)


def kernel(x, edge_index, W1, b1, W2, b2):
    n_nodes = x.shape[0]
    n_edges = edge_index.shape[1]
    nvalid_ps = n_edges // N_SUB
    nck = -(-nvalid_ps // CHUNK)
    nck += nck % 2
    npad = nck * CHUNK - nvalid_ps
    src = jnp.concatenate(
        [edge_index[0].reshape(N_SUB, nvalid_ps),
         jnp.zeros((N_SUB, npad), jnp.int32)], axis=1).reshape(N_SUB, nck, CHUNK)
    dst = jnp.concatenate(
        [edge_index[1].reshape(N_SUB, nvalid_ps),
         jnp.full((N_SUB, npad), n_nodes, jnp.int32)], axis=1).reshape(N_SUB, nck, CHUNK)

    h1s = _linear_split(x, W1, b1)
    acc, deg = _edge_aggregate(h1s, src, dst, nvalid_ps, n_nodes)
    return _combine(acc, deg.reshape(2 * N_SUB, n_nodes), W2, b2, n_edges)


# speedup vs baseline: 1.6400x; 1.6400x over previous
import functools

import jax
import jax.numpy as jnp
from jax import lax
from jax.experimental import pallas as pl
from jax.experimental.pallas import tpu as pltpu
from jax.experimental.pallas import tpu_sc as plsc

LANES = 16
CHUNK = 80
HALF = 64
N_SUB = 16


def _linear_body(x_ref, w_ref, b_ref, o_ref):
    h = lax.dot_general(
        x_ref[...], w_ref[...], (((1,), (1,)), ((), ())),
        preferred_element_type=jnp.float32) + b_ref[...][None, :]
    o_ref[0] = h[:, :HALF]
    o_ref[1] = h[:, HALF:]


def _linear_split(x, W, b):
    n, _ = x.shape
    return pl.pallas_call(
        _linear_body,
        out_shape=jax.ShapeDtypeStruct((2, n, HALF), jnp.float32),
    )(x, W, b)


def _edge_agg_body(nchunks_ps, n_nodes,
                   h1_hbm, srcm_hbm, dstm_hbm, acc_hbm, deg_hbm,
                   r0, r1, r2, r3, r4, src_v, dst_v, zbuf_v, degl_v,
                   acc_sh, s0, s1, s2, s3, s4):
    rows_v = [r0, r1, r2, r3, r4]
    semg = [s0, s1, s2, s3, s4]
    cid = lax.axis_index("c")
    sid = lax.axis_index("s")
    zrows = zbuf_v.shape[0]
    nzchunks = n_nodes // zrows
    zk = (nzchunks + N_SUB - 1) // N_SUB
    nbuf = len(rows_v)

    ds = pltpu.async_copy(srcm_hbm.at[sid], src_v, semg[0])
    dd = pltpu.async_copy(dstm_hbm.at[sid], dst_v, semg[1])

    def _z2(i, _):
        r = i // (HALF // LANES)
        c = i % (HALF // LANES)
        zbuf_v[r, pl.ds(c * LANES, LANES)] = jnp.zeros((LANES,), jnp.float32)
        return 0
    lax.fori_loop(0, zrows * (HALF // LANES), _z2, 0)

    def _zd(i, _):
        degl_v[pl.ds(i * LANES, LANES)] = jnp.zeros((LANES,), jnp.float32)
        return 0
    lax.fori_loop(0, n_nodes // LANES, _zd, 0)

    ds.wait()
    dd.wait()

    def _zacc(k, _):
        j = sid + k * N_SUB
        @pl.when(j < nzchunks)
        def _():
            pltpu.sync_copy(zbuf_v, acc_sh.at[pl.ds(j * zrows, zrows)])
        return 0
    lax.fori_loop(0, zk, _zacc, 0)

    plsc.subcore_barrier()

    table = h1_hbm.at[cid]
    ones16 = jnp.ones((LANES,), jnp.float32)
    vpc = CHUNK // LANES
    for k in range(nbuf - 2):
        pltpu.async_copy(table.at[src_v.at[k]], rows_v[k], semg[k])

    def _edge(i, _):
        j0 = nbuf * i
        for k in range(nbuf):
            j = j0 + k
            pltpu.make_async_copy(table.at[src_v.at[j]], rows_v[k], semg[k]).wait()
            jn = jnp.where(j + nbuf - 2 < nchunks_ps, j + nbuf - 2, 0)
            pltpu.async_copy(table.at[src_v.at[jn]], rows_v[(k + nbuf - 2) % nbuf],
                             semg[(k + nbuf - 2) % nbuf])
            for c in range(vpc):
                idx = src_v[j, pl.ds(c * LANES, LANES)]
                plsc.addupdate_scatter(degl_v, [idx], ones16)
            pltpu.sync_copy(rows_v[k], acc_sh.at[dst_v.at[j]], add=True)
        return 0
    lax.fori_loop(0, nchunks_ps // nbuf, _edge, 0)
    for k in range(nbuf - 2):
        pltpu.make_async_copy(table.at[src_v.at[0]], rows_v[k], semg[k]).wait()

    pltpu.sync_copy(degl_v,
                    deg_hbm.at[pl.ds((cid * N_SUB + sid) * n_nodes, n_nodes)])

    plsc.subcore_barrier()

    def _wacc(k, _):
        j = sid + k * N_SUB
        @pl.when(j < nzchunks)
        def _():
            off = j * zrows
            pltpu.sync_copy(acc_sh.at[pl.ds(off, zrows)], zbuf_v)
            pltpu.sync_copy(zbuf_v, acc_hbm.at[cid, pl.ds(off, zrows)])
        return 0
    lax.fori_loop(0, zk, _wacc, 0)


def _edge_aggregate(h1s, srcm, dstm, n_nodes):
    nchunks_ps = srcm.shape[1]
    mesh = plsc.VectorSubcoreMesh(core_axis_name="c", subcore_axis_name="s")
    kern = pl.kernel(
        functools.partial(_edge_agg_body, nchunks_ps, n_nodes),
        out_type=(
            jax.ShapeDtypeStruct((2, n_nodes, HALF), jnp.float32),
            jax.ShapeDtypeStruct((2 * N_SUB * n_nodes,), jnp.float32),
        ),
        mesh=mesh,
        compiler_params=pltpu.CompilerParams(use_tc_tiling_on_sc=False,
                                             needs_layout_passes=False),
        scratch_types=(
            pltpu.VMEM((CHUNK, HALF), jnp.float32),
            pltpu.VMEM((CHUNK, HALF), jnp.float32),
            pltpu.VMEM((CHUNK, HALF), jnp.float32),
            pltpu.VMEM((CHUNK, HALF), jnp.float32),
            pltpu.VMEM((CHUNK, HALF), jnp.float32),
            pltpu.VMEM((nchunks_ps, CHUNK), jnp.int32),
            pltpu.VMEM((nchunks_ps, CHUNK), jnp.int32),
            pltpu.VMEM((200, HALF), jnp.float32),
            pltpu.VMEM((n_nodes,), jnp.float32),
            pltpu.VMEM_SHARED((n_nodes, HALF), jnp.float32),
            pltpu.SemaphoreType.DMA,
            pltpu.SemaphoreType.DMA,
            pltpu.SemaphoreType.DMA,
            pltpu.SemaphoreType.DMA,
            pltpu.SemaphoreType.DMA,
        ),
    )
    return kern(h1s, srcm, dstm)


def _combine_body(n_edges, acc_ref, deg_ref, w2_ref, b2_ref, o_ref):
    w2s = jnp.sum(w2_ref[...], axis=0)
    deg = 0.5 * jnp.sum(deg_ref[...], axis=0)
    ta = jnp.tanh(acc_ref[0])
    tb = jnp.tanh(acc_ref[1])
    row = jnp.sum(ta * w2s[None, :HALF], axis=1) + jnp.sum(tb * w2s[None, HALF:], axis=1)
    total = jnp.sum(row * deg) + n_edges * jnp.sum(b2_ref[...])
    o_ref[...] = total[None, None]


def _combine(acc, deg, W2, b2, n_edges):
    return pl.pallas_call(
        functools.partial(_combine_body, float(n_edges)),
        out_shape=jax.ShapeDtypeStruct((1, 1), jnp.float32),
    )(acc, deg, W2, b2)


def kernel(x, edge_index, W1, b1, W2, b2):
    n_nodes = x.shape[0]
    n_edges = edge_index.shape[1]
    src = edge_index[0].reshape(N_SUB, n_edges // (N_SUB * CHUNK), CHUNK)
    dst = edge_index[1].reshape(N_SUB, n_edges // (N_SUB * CHUNK), CHUNK)

    h1s = _linear_split(x, W1, b1)
    acc, deg = _edge_aggregate(h1s, src, dst, n_nodes)
    return _combine(acc, deg.reshape(2 * N_SUB, n_nodes), W2, b2, n_edges)


# speedup vs baseline: 1.6755x; 1.0216x over previous
import functools

import jax
import jax.numpy as jnp
from jax import lax
from jax.experimental import pallas as pl
from jax.experimental.pallas import tpu as pltpu
from jax.experimental.pallas import tpu_sc as plsc

LANES = 16
CHUNK = 80
HALF = 64
N_SUB = 16


def _linear_body(x_ref, w_ref, b_ref, o_ref):
    h = lax.dot_general(
        x_ref[...], w_ref[...], (((1,), (1,)), ((), ())),
        preferred_element_type=jnp.float32) + b_ref[...][None, :]
    o_ref[0] = h[:, :HALF]
    o_ref[1] = h[:, HALF:]


def _linear_split(x, W, b):
    n, _ = x.shape
    return pl.pallas_call(
        _linear_body,
        out_shape=jax.ShapeDtypeStruct((2, n, HALF), jnp.float32),
    )(x, W, b)


def _edge_agg_body(nchunks_ps, n_nodes,
                   h1_hbm, srcm_hbm, dstm_hbm, acc_hbm, deg_hbm,
                   r0, r1, r2, r3, r4, src_v, dst_v, zbuf_v, degl_v,
                   acc_sh, s0, s1, s2, s3, s4):
    rows_v = [r0, r1, r2, r3, r4]
    semg = [s0, s1, s2, s3, s4]
    cid = lax.axis_index("c")
    sid = lax.axis_index("s")
    zrows = zbuf_v.shape[0]
    nzchunks = n_nodes // zrows
    zk = (nzchunks + N_SUB - 1) // N_SUB
    nbuf = len(rows_v)
    nfly = nbuf - 1

    ds = pltpu.async_copy(srcm_hbm.at[sid], src_v, semg[0])
    dd = pltpu.async_copy(dstm_hbm.at[sid], dst_v, semg[1])

    def _z2(i, _):
        r = i // (HALF // LANES)
        c = i % (HALF // LANES)
        zbuf_v[r, pl.ds(c * LANES, LANES)] = jnp.zeros((LANES,), jnp.float32)
        return 0
    lax.fori_loop(0, zrows * (HALF // LANES), _z2, 0)

    def _zd(i, _):
        degl_v[pl.ds(i * LANES, LANES)] = jnp.zeros((LANES,), jnp.float32)
        return 0
    lax.fori_loop(0, n_nodes // LANES, _zd, 0)

    ds.wait()
    dd.wait()

    def _zacc(k, _):
        j = sid + k * N_SUB
        @pl.when(j < nzchunks)
        def _():
            pltpu.sync_copy(zbuf_v, acc_sh.at[pl.ds(j * zrows, zrows)])
        return 0
    lax.fori_loop(0, zk, _zacc, 0)

    plsc.subcore_barrier()

    table = h1_hbm.at[cid]
    ones16 = jnp.ones((LANES,), jnp.float32)
    vpc = CHUNK // LANES
    for k in range(nfly):
        pltpu.async_copy(table.at[src_v.at[k]], rows_v[k], semg[k])

    def _edge(i, _):
        j0 = nbuf * i
        for k in range(nbuf):
            j = j0 + k
            pltpu.make_async_copy(table.at[src_v.at[j]], rows_v[k], semg[k]).wait()
            jn = jnp.where(j + nfly < nchunks_ps, j + nfly, 0)
            pltpu.async_copy(table.at[src_v.at[jn]], rows_v[(k + nfly) % nbuf],
                             semg[(k + nfly) % nbuf])
            for c in range(vpc):
                idx = src_v[j, pl.ds(c * LANES, LANES)]
                plsc.addupdate_scatter(degl_v, [idx], ones16)
            pltpu.sync_copy(rows_v[k], acc_sh.at[dst_v.at[j]], add=True)
        return 0
    lax.fori_loop(0, nchunks_ps // nbuf, _edge, 0)
    for k in range(nfly):
        pltpu.make_async_copy(table.at[src_v.at[0]], rows_v[k], semg[k]).wait()

    pltpu.sync_copy(degl_v,
                    deg_hbm.at[pl.ds((cid * N_SUB + sid) * n_nodes, n_nodes)])

    plsc.subcore_barrier()

    def _wacc(k, _):
        j = sid + k * N_SUB
        @pl.when(j < nzchunks)
        def _():
            off = j * zrows
            pltpu.sync_copy(acc_sh.at[pl.ds(off, zrows)], zbuf_v)
            pltpu.sync_copy(zbuf_v, acc_hbm.at[cid, pl.ds(off, zrows)])
        return 0
    lax.fori_loop(0, zk, _wacc, 0)


def _edge_aggregate(h1s, srcm, dstm, n_nodes):
    nchunks_ps = srcm.shape[1]
    mesh = plsc.VectorSubcoreMesh(core_axis_name="c", subcore_axis_name="s")
    kern = pl.kernel(
        functools.partial(_edge_agg_body, nchunks_ps, n_nodes),
        out_type=(
            jax.ShapeDtypeStruct((2, n_nodes, HALF), jnp.float32),
            jax.ShapeDtypeStruct((2 * N_SUB * n_nodes,), jnp.float32),
        ),
        mesh=mesh,
        compiler_params=pltpu.CompilerParams(use_tc_tiling_on_sc=False,
                                             needs_layout_passes=False),
        scratch_types=(
            pltpu.VMEM((CHUNK, HALF), jnp.float32),
            pltpu.VMEM((CHUNK, HALF), jnp.float32),
            pltpu.VMEM((CHUNK, HALF), jnp.float32),
            pltpu.VMEM((CHUNK, HALF), jnp.float32),
            pltpu.VMEM((CHUNK, HALF), jnp.float32),
            pltpu.VMEM((nchunks_ps, CHUNK), jnp.int32),
            pltpu.VMEM((nchunks_ps, CHUNK), jnp.int32),
            pltpu.VMEM((200, HALF), jnp.float32),
            pltpu.VMEM((n_nodes,), jnp.float32),
            pltpu.VMEM_SHARED((n_nodes, HALF), jnp.float32),
            pltpu.SemaphoreType.DMA,
            pltpu.SemaphoreType.DMA,
            pltpu.SemaphoreType.DMA,
            pltpu.SemaphoreType.DMA,
            pltpu.SemaphoreType.DMA,
        ),
    )
    return kern(h1s, srcm, dstm)


def _combine_body(n_edges, acc_ref, deg_ref, w2_ref, b2_ref, o_ref):
    w2s = jnp.sum(w2_ref[...], axis=0)
    deg = 0.5 * jnp.sum(deg_ref[...], axis=0)
    ta = jnp.tanh(acc_ref[0])
    tb = jnp.tanh(acc_ref[1])
    row = jnp.sum(ta * w2s[None, :HALF], axis=1) + jnp.sum(tb * w2s[None, HALF:], axis=1)
    total = jnp.sum(row * deg) + n_edges * jnp.sum(b2_ref[...])
    o_ref[...] = total[None, None]


def _combine(acc, deg, W2, b2, n_edges):
    return pl.pallas_call(
        functools.partial(_combine_body, float(n_edges)),
        out_shape=jax.ShapeDtypeStruct((1, 1), jnp.float32),
    )(acc, deg, W2, b2)


def kernel(x, edge_index, W1, b1, W2, b2):
    n_nodes = x.shape[0]
    n_edges = edge_index.shape[1]
    src = edge_index[0].reshape(N_SUB, n_edges // (N_SUB * CHUNK), CHUNK)
    dst = edge_index[1].reshape(N_SUB, n_edges // (N_SUB * CHUNK), CHUNK)

    h1s = _linear_split(x, W1, b1)
    acc, deg = _edge_aggregate(h1s, src, dst, n_nodes)
    return _combine(acc, deg.reshape(2 * N_SUB, n_nodes), W2, b2, n_edges)
